# baseline (speedup 1.0000x reference)
import jax
import jax.numpy as jnp
from jax import lax
from jax.experimental import pallas as pl
from jax.experimental.pallas import tpu as pltpu
from jax.experimental.pallas import tpu_sc as plsc

B = 16384
T = 50
D = 64
NC = 2
NS = 16
NW = NC * NS
B_PER_W = B // NW
GB = 8
N_GROUPS = B_PER_W // GB
NBUF = 3
MAIN = ((N_GROUPS - NBUF) // NBUF) * NBUF


def _sc_body(idx_hbm, table_hbm, out_hbm, idx_v, rows0, rows1, rows2,
             sg0, sg1, sg2, ss0, ss1, ss2):
  wid = lax.axis_index("s") * NC + lax.axis_index("c")
  base = wid * B_PER_W
  rows = (rows0, rows1, rows2)
  sg = (sg0, sg1, sg2)
  ss = (ss0, ss1, ss2)

  pltpu.sync_copy(idx_hbm.at[pl.ds(base, B_PER_W)], idx_v)

  def fire_gathers(g, b):
    for k in range(GB):
      pltpu.async_copy(
          table_hbm.at[idx_v.at[g * GB + k]],
          rows[b].at[k],
          sg[b])

  def wait_gathers(b):
    pltpu.make_async_copy(out_hbm.at[pl.ds(base, GB)], rows[b], sg[b]).wait()

  def fire_scatter(g, b):
    pltpu.async_copy(rows[b], out_hbm.at[pl.ds(base + g * GB, GB)], ss[b])

  def wait_scatter(b):
    pltpu.make_async_copy(rows[b], out_hbm.at[pl.ds(base, GB)], ss[b]).wait()

  for b in range(NBUF):
    fire_gathers(b, b)

  @pl.loop(0, MAIN, step=NBUF)
  def _(t):
    for b in range(NBUF):
      g = t + b
      wait_gathers(b)
      fire_scatter(g, b)
      wait_scatter(b)
      fire_gathers(g + NBUF, b)

  for g in range(MAIN, N_GROUPS):
    b = g % NBUF
    wait_gathers(b)
    fire_scatter(g, b)
    if g + NBUF < N_GROUPS:
      wait_scatter(b)
      fire_gathers(g + NBUF, b)
  for g in range(N_GROUPS - NBUF, N_GROUPS):
    wait_scatter(g % NBUF)


_lookup = pl.kernel(
    _sc_body,
    out_type=jax.ShapeDtypeStruct((B, T, D), jnp.float32),
    mesh=plsc.VectorSubcoreMesh(core_axis_name="c", subcore_axis_name="s"),
    scratch_types=[
        pltpu.VMEM((B_PER_W, T), jnp.int32),
        pltpu.VMEM((GB, T, D), jnp.float32),
        pltpu.VMEM((GB, T, D), jnp.float32),
        pltpu.VMEM((GB, T, D), jnp.float32),
        pltpu.SemaphoreType.DMA,
        pltpu.SemaphoreType.DMA,
        pltpu.SemaphoreType.DMA,
        pltpu.SemaphoreType.DMA,
        pltpu.SemaphoreType.DMA,
        pltpu.SemaphoreType.DMA,
    ],
    compiler_params=pltpu.CompilerParams(use_tc_tiling_on_sc=False),
)


@jax.jit
def kernel(action_ids, embedding_weight):
  return _lookup(action_ids.astype(jnp.int32), embedding_weight)


# speedup vs baseline: 1.0931x; 1.0931x over previous
import jax
import jax.numpy as jnp
from jax import lax
from jax.experimental import pallas as pl
from jax.experimental.pallas import tpu as pltpu
from jax.experimental.pallas import tpu_sc as plsc

B = 16384
T = 50
D = 64
DP = 128
VL = 16
NC = 2
NS = 16
NW = NC * NS
B_PER_W = B // NW
CB = 128
N_CHUNKS = B_PER_W // CB
GB = 4
N_GROUPS = CB // GB
NBUF = 2


def _sc_body(idx_hbm, table_hbm, out_hbm,
             idx_v, raw0, raw1, cmp0, cmp1, sg0, sg1, ss0, ss1):
  wid = lax.axis_index("s") * NC + lax.axis_index("c")
  base = wid * B_PER_W
  raw = (raw0, raw1)
  cmp = (cmp0, cmp1)
  sg = (sg0, sg1)
  ss = (ss0, ss1)

  @pl.loop(0, N_CHUNKS)
  def _(c):
    cbase = base + c * CB
    pltpu.sync_copy(idx_hbm.at[pl.ds(cbase, CB)], idx_v)

    gh = [None] * NBUF
    sh = [None] * NBUF

    def fire_gathers(g, b):
      gh[b] = [pltpu.async_copy(table_hbm.at[idx_v.at[g * GB + k]],
                                raw[b].at[k], sg[b])
               for k in range(GB)]

    def wait_gathers(b):
      for h in gh[b]:
        h.wait()

    def compact(b):
      @pl.loop(0, T)
      def _(t):
        for k in range(GB):
          for h in range(D // VL):
            cmp[b][k, t, pl.ds(h * VL, VL)] = raw[b][k, t, pl.ds(h * VL, VL)]

    def fire_scatters(g, b):
      sh[b] = [pltpu.async_copy(cmp[b].at[k],
                                out_hbm.at[cbase + g * GB + k], ss[b])
               for k in range(GB)]

    def wait_scatters(b):
      for h in sh[b]:
        h.wait()

    for b in range(NBUF):
      fire_gathers(b, b)
    for g in range(N_GROUPS):
      b = g % NBUF
      wait_gathers(b)
      compact(b)
      fire_scatters(g, b)
      if g + NBUF < N_GROUPS:
        wait_scatters(b)
        fire_gathers(g + NBUF, b)
    for g in range(N_GROUPS - NBUF, N_GROUPS):
      wait_scatters(g % NBUF)


_lookup = pl.kernel(
    _sc_body,
    out_type=jax.ShapeDtypeStruct((B, T, D), jnp.float32),
    mesh=plsc.VectorSubcoreMesh(core_axis_name="c", subcore_axis_name="s"),
    scratch_types=[
        pltpu.VMEM((CB, T), jnp.int32),
        pltpu.VMEM((GB, T, DP), jnp.float32),
        pltpu.VMEM((GB, T, DP), jnp.float32),
        pltpu.VMEM((GB, T, D), jnp.float32),
        pltpu.VMEM((GB, T, D), jnp.float32),
        pltpu.SemaphoreType.DMA,
        pltpu.SemaphoreType.DMA,
        pltpu.SemaphoreType.DMA,
        pltpu.SemaphoreType.DMA,
    ],
    compiler_params=pltpu.CompilerParams(use_tc_tiling_on_sc=True),
)


@jax.jit
def kernel(action_ids, embedding_weight):
  padded = jnp.pad(embedding_weight, ((0, 0), (0, DP - D)))
  return _lookup(action_ids.astype(jnp.int32), padded)
